# Initial kernel scaffold; baseline (speedup 1.0000x reference)
import jax
import jax.numpy as jnp
from jax.experimental import pallas as pl


def kernel(x, proj_range_xyz, unproj_range_xyz, p2ri_lut, num_valid_pts, fc1_w, fc1_b, ln_w, ln_b, fc2_w, fc2_b, conv_w, conv_b):
    raise NotImplementedError("write your pallas kernel here")


# speedup vs baseline: 22.6223x; 22.6223x over previous
import functools

import jax
import jax.numpy as jnp
from jax import lax
from jax.experimental import pallas as pl
from jax.experimental.pallas import tpu as pltpu
from jax.experimental.pallas import tpu_sc as plsc

_NCLASSES = 19
_S = 7
_K = _S * _S
_CX = 32
_CP = 16
_H, _W = 64, 2048
_PAD = (_S - 1) // 2
_HP, _WP = _H + 2 * _PAD, _W + 2 * _PAD
_N = 16384
_ROWS = _K * _N
_IDX_MINOR = 98
_IDX_MAJOR = _ROWS // _IDX_MINOR
_CHUNK_GROUPS = 8
_CHUNK = _CHUNK_GROUPS * _IDX_MINOR
_NTILES = 32
_ROWS_PER_TILE = _ROWS // _NTILES
_CHUNKS_PER_TILE = _ROWS_PER_TILE // _CHUNK

_PBLK = 128
_GRID = _N // _PBLK


def _sc_gather(idx2, tx, tp):
    mesh = plsc.VectorSubcoreMesh(core_axis_name="c", subcore_axis_name="s")

    @functools.partial(
        pl.kernel,
        mesh=mesh,
        compiler_params=pltpu.CompilerParams(use_tc_tiling_on_sc=False),
        out_type=[
            jax.ShapeDtypeStruct((_ROWS, _CX), jnp.float32),
            jax.ShapeDtypeStruct((_ROWS, _CP), jnp.float32),
        ],
        scratch_types=[
            pltpu.VMEM((_CHUNK_GROUPS, _IDX_MINOR), jnp.int32),
            pltpu.VMEM((_CHUNK, _CX), jnp.float32),
            pltpu.VMEM((_CHUNK, _CP), jnp.float32),
            pltpu.SemaphoreType.DMA,
        ],
    )
    def k(idx_hbm, tx_hbm, tp_hbm, gx_hbm, gp_hbm, idx_v, gx_v, gp_v, sem):
        wid = lax.axis_index("s") * 2 + lax.axis_index("c")
        idx_row0 = wid * (_ROWS_PER_TILE // _IDX_MINOR)
        row0 = wid * _ROWS_PER_TILE

        def body(c, _):
            pltpu.sync_copy(
                idx_hbm.at[pl.ds(idx_row0 + c * _CHUNK_GROUPS, _CHUNK_GROUPS)],
                idx_v)
            handles = []
            for j in range(_CHUNK_GROUPS):
                handles.append(pltpu.async_copy(
                    tx_hbm.at[idx_v.at[j]],
                    gx_v.at[pl.ds(j * _IDX_MINOR, _IDX_MINOR)], sem))
                handles.append(pltpu.async_copy(
                    tp_hbm.at[idx_v.at[j]],
                    gp_v.at[pl.ds(j * _IDX_MINOR, _IDX_MINOR)], sem))
            for h in handles:
                h.wait()
            base = row0 + c * _CHUNK
            pltpu.sync_copy(gx_v, gx_hbm.at[pl.ds(base, _CHUNK)])
            pltpu.sync_copy(gp_v, gp_hbm.at[pl.ds(base, _CHUNK)])
            return _

        lax.fori_loop(0, _CHUNKS_PER_TILE, body, None)

    return k(idx2, tx, tp)


def _tc_body(gx_ref, gp_ref, up_ref, fc1_ref, b1_ref, lnw_ref, lnb_ref,
             fc2_ref, b2_ref, conv_ref, cb_ref, o64_ref, o32_ref, out_ref):
    gp3 = gp_ref[...]
    up = up_ref[...]
    d = jnp.abs(gp3 - up[None, :, :])
    d2 = d.reshape(_K * _PBLK, _CP)
    hc = jnp.dot(d2, fc1_ref[...], preferred_element_type=jnp.float32) + b1_ref[...]
    vs = jnp.dot(hc * hc, o64_ref[...], preferred_element_type=jnp.float32)
    inv = lax.rsqrt(vs[:, :1] + 1e-5)
    h = hc * inv * lnw_ref[...] + lnb_ref[...]
    h = 0.5 * h * (1.0 + lax.erf(h * 0.7071067811865476))
    h = jnp.dot(h, fc2_ref[...], preferred_element_type=jnp.float32) + b2_ref[...]
    e = jnp.exp(h)
    s = jnp.dot(e, o32_ref[...], preferred_element_type=jnp.float32)
    w = e * (1.0 / s[:, :1])
    wx = gx_ref[...].reshape(_K * _PBLK, _CX) * w
    wx3 = wx.reshape(_K, _PBLK, _CX)
    resT = lax.dot_general(
        conv_ref[...], wx3,
        dimension_numbers=(((1,), (2,)), ((0,), (0,))),
        preferred_element_type=jnp.float32)
    out_ref[...] = jnp.sum(resT, axis=0) + cb_ref[...][:, None]


def _tc_mlp(gx3, gp3, up16, fc1c, b1c, ln_w, ln_b, fc2t, fc2_b, convr3, conv_b,
            o64, o32):
    return pl.pallas_call(
        _tc_body,
        grid=(_GRID,),
        in_specs=[
            pl.BlockSpec((_K, _PBLK, _CX), lambda i: (0, i, 0)),
            pl.BlockSpec((_K, _PBLK, _CP), lambda i: (0, i, 0)),
            pl.BlockSpec((_PBLK, _CP), lambda i: (i, 0)),
            pl.BlockSpec((_CP, 64), lambda i: (0, 0)),
            pl.BlockSpec((64,), lambda i: (0,)),
            pl.BlockSpec((64,), lambda i: (0,)),
            pl.BlockSpec((64,), lambda i: (0,)),
            pl.BlockSpec((64, _CX), lambda i: (0, 0)),
            pl.BlockSpec((_CX,), lambda i: (0,)),
            pl.BlockSpec((_K, _CX, _NCLASSES), lambda i: (0, 0, 0)),
            pl.BlockSpec((_NCLASSES,), lambda i: (0,)),
            pl.BlockSpec((64, 8), lambda i: (0, 0)),
            pl.BlockSpec((_CX, 8), lambda i: (0, 0)),
        ],
        out_specs=pl.BlockSpec((_NCLASSES, _PBLK), lambda i: (0, i)),
        out_shape=jax.ShapeDtypeStruct((_NCLASSES, _N), jnp.float32),
    )(gx3, gp3, up16, fc1c, b1c, ln_w, ln_b, fc2t, fc2_b, convr3, conv_b,
      o64, o32)


def kernel(x, proj_range_xyz, unproj_range_xyz, p2ri_lut, num_valid_pts,
           fc1_w, fc1_b, ln_w, ln_b, fc2_w, fc2_b, conv_w, conv_b):
    xp = jnp.pad(x[0], ((0, 0), (_PAD, _PAD), (_PAD, _PAD)))
    pp = jnp.pad(proj_range_xyz[0], ((0, 0), (_PAD, _PAD), (_PAD, _PAD)))
    tx = jnp.transpose(xp, (1, 2, 0)).reshape(_HP * _WP, _CX)
    tp = jnp.transpose(pp, (1, 2, 0))
    tp = jnp.pad(tp, ((0, 0), (0, 0), (0, _CP - 4))).reshape(_HP * _WP, _CP)

    lut = p2ri_lut[0]
    yc = lut[:, 1]
    xc = lut[:, 2]
    dy = jnp.arange(_S, dtype=jnp.int32)
    off = (dy[:, None] * _WP + dy[None, :]).reshape(_K)
    base = yc * _WP + xc
    idx = (off[:, None] + base[None, :]).reshape(_ROWS)
    idx2 = idx.reshape(_IDX_MAJOR, _IDX_MINOR).astype(jnp.int32)

    up16 = jnp.pad(unproj_range_xyz[0], ((0, 0), (0, _CP - 4)))
    fc1p = jnp.pad(fc1_w.T, ((0, _CP - 4), (0, 0)))
    fc1c = fc1p - jnp.mean(fc1p, axis=1, keepdims=True)
    b1c = fc1_b - jnp.mean(fc1_b)
    fc2t = fc2_w.T
    convr3 = jnp.transpose(conv_w.reshape(_NCLASSES, _CX, _K), (2, 1, 0))
    o64 = jnp.full((64, 8), 1.0 / 64, jnp.float32)
    o32 = jnp.ones((_CX, 8), jnp.float32)

    gx, gp = _sc_gather(idx2, tx, tp)
    gx3 = gx.reshape(_K, _N, _CX)
    gp3 = gp.reshape(_K, _N, _CP)

    out = _tc_mlp(gx3, gp3, up16, fc1c, b1c, ln_w, ln_b, fc2t, fc2_b,
                  convr3, conv_b, o64, o32)
    return out[None]


# speedup vs baseline: 61.7996x; 2.7318x over previous
import functools

import jax
import jax.numpy as jnp
from jax import lax
from jax.experimental import pallas as pl
from jax.experimental.pallas import tpu as pltpu
from jax.experimental.pallas import tpu_sc as plsc

_NCLASSES = 19
_S = 7
_K = _S * _S
_C = 32
_H, _W = 64, 2048
_PAD = (_S - 1) // 2
_HP, _WP = _H + 2 * _PAD, _W + 2 * _PAD
_N = 16384
_ROWS = _K * _N
_IDX_MINOR = 98
_IDX_MAJOR = _ROWS // _IDX_MINOR
_CHUNK_GROUPS = 8
_CHUNK = _CHUNK_GROUPS * _IDX_MINOR
_NTILES = 32
_ROWS_PER_TILE = _ROWS // _NTILES
_CHUNKS_PER_TILE = _ROWS_PER_TILE // _CHUNK

_PPACK = 4
_Q = _N // _PPACK
_QBLK = 64
_GRID = _Q // _QBLK
_R = _K * _QBLK
_OL = _PPACK * _NCLASSES


def _sc_gather(idx2, tx, tp):
    mesh = plsc.VectorSubcoreMesh(core_axis_name="c", subcore_axis_name="s")

    @functools.partial(
        pl.kernel,
        mesh=mesh,
        compiler_params=pltpu.CompilerParams(use_tc_tiling_on_sc=False),
        out_type=[
            jax.ShapeDtypeStruct((_ROWS, _C), jnp.float32),
            jax.ShapeDtypeStruct((_ROWS, _C), jnp.float32),
        ],
        scratch_types=[
            pltpu.VMEM((_CHUNK_GROUPS, _IDX_MINOR), jnp.int32),
            pltpu.VMEM((_CHUNK, _C), jnp.float32),
            pltpu.VMEM((_CHUNK, _C), jnp.float32),
            pltpu.SemaphoreType.DMA,
        ],
    )
    def k(idx_hbm, tx_hbm, tp_hbm, gx_hbm, gp_hbm, idx_v, gx_v, gp_v, sem):
        wid = lax.axis_index("s") * 2 + lax.axis_index("c")
        idx_row0 = wid * (_ROWS_PER_TILE // _IDX_MINOR)
        row0 = wid * _ROWS_PER_TILE

        def body(c, _):
            pltpu.sync_copy(
                idx_hbm.at[pl.ds(idx_row0 + c * _CHUNK_GROUPS, _CHUNK_GROUPS)],
                idx_v)
            handles = []
            for j in range(_CHUNK_GROUPS):
                handles.append(pltpu.async_copy(
                    tx_hbm.at[idx_v.at[j]],
                    gx_v.at[pl.ds(j * _IDX_MINOR, _IDX_MINOR)], sem))
                handles.append(pltpu.async_copy(
                    tp_hbm.at[idx_v.at[j]],
                    gp_v.at[pl.ds(j * _IDX_MINOR, _IDX_MINOR)], sem))
            for h in handles:
                h.wait()
            base = row0 + c * _CHUNK
            pltpu.sync_copy(gx_v, gx_hbm.at[pl.ds(base, _CHUNK)])
            pltpu.sync_copy(gp_v, gp_hbm.at[pl.ds(base, _CHUNK)])
            return _

        lax.fori_loop(0, _CHUNKS_PER_TILE, body, None)

    return k(idx2, tx, tp)


def _tc_body(gx_ref, gp_ref, up_ref, fc1_ref, b1_ref, lnw_ref, lnb_ref,
             g_ref, bg_ref, fc2_ref, b2_ref, s_ref, bs_ref, conv_ref, cb_ref,
             out_ref):
    gp3 = gp_ref[...]
    up = up_ref[...]
    d = jnp.abs(gp3 - up[None, :, :]).reshape(_R, 128)
    hc = jnp.dot(d, fc1_ref[...], preferred_element_type=jnp.float32) + b1_ref[...]
    vs = jnp.dot(hc * hc, g_ref[...], preferred_element_type=jnp.float32)
    inv = jnp.dot(lax.rsqrt(vs + 1e-5), bg_ref[...],
                  preferred_element_type=jnp.float32)
    h = hc * inv * lnw_ref[...] + lnb_ref[...]
    h = 0.5 * h * (1.0 + lax.erf(h * 0.7071067811865476))
    h = jnp.dot(h, fc2_ref[...], preferred_element_type=jnp.float32) + b2_ref[...]
    e = jnp.exp(h)
    s = jnp.dot(e, s_ref[...], preferred_element_type=jnp.float32)
    w = e * jnp.dot(1.0 / s, bs_ref[...], preferred_element_type=jnp.float32)
    wx = gx_ref[...].reshape(_R, 128) * w
    wx3 = wx.reshape(_K, _QBLK, 128)
    res = lax.dot_general(
        wx3, conv_ref[...],
        dimension_numbers=(((2,), (1,)), ((0,), (0,))),
        preferred_element_type=jnp.float32)
    out_ref[...] = jnp.sum(res, axis=0) + cb_ref[...]


def _tc_mlp(gx3, gp3, up128, fc1bd, b1c, lnw, lnb, gmat, bg, fc2bd, b2t,
            smat, bs, conv4, cb):
    return pl.pallas_call(
        _tc_body,
        grid=(_GRID,),
        in_specs=[
            pl.BlockSpec((_K, _QBLK, 128), lambda i: (0, i, 0)),
            pl.BlockSpec((_K, _QBLK, 128), lambda i: (0, i, 0)),
            pl.BlockSpec((_QBLK, 128), lambda i: (i, 0)),
            pl.BlockSpec((128, 256), lambda i: (0, 0)),
            pl.BlockSpec((256,), lambda i: (0,)),
            pl.BlockSpec((256,), lambda i: (0,)),
            pl.BlockSpec((256,), lambda i: (0,)),
            pl.BlockSpec((256, _PPACK), lambda i: (0, 0)),
            pl.BlockSpec((_PPACK, 256), lambda i: (0, 0)),
            pl.BlockSpec((256, 128), lambda i: (0, 0)),
            pl.BlockSpec((128,), lambda i: (0,)),
            pl.BlockSpec((128, _PPACK), lambda i: (0, 0)),
            pl.BlockSpec((_PPACK, 128), lambda i: (0, 0)),
            pl.BlockSpec((_K, 128, _OL), lambda i: (0, 0, 0)),
            pl.BlockSpec((1, _OL), lambda i: (0, 0)),
        ],
        out_specs=pl.BlockSpec((_QBLK, _OL), lambda i: (i, 0)),
        out_shape=jax.ShapeDtypeStruct((_Q, _OL), jnp.float32),
    )(gx3, gp3, up128, fc1bd, b1c, lnw, lnb, gmat, bg, fc2bd, b2t,
      smat, bs, conv4, cb)


def kernel(x, proj_range_xyz, unproj_range_xyz, p2ri_lut, num_valid_pts,
           fc1_w, fc1_b, ln_w, ln_b, fc2_w, fc2_b, conv_w, conv_b):
    f32 = jnp.float32
    xp = jnp.pad(x[0], ((0, 0), (_PAD, _PAD), (_PAD, _PAD)))
    pp = jnp.pad(proj_range_xyz[0], ((0, 0), (_PAD, _PAD), (_PAD, _PAD)))
    tx = jnp.transpose(xp, (1, 2, 0)).reshape(_HP * _WP, _C)
    tp = jnp.transpose(pp, (1, 2, 0))
    tp = jnp.pad(tp, ((0, 0), (0, 0), (0, _C - 4))).reshape(_HP * _WP, _C)

    lut = p2ri_lut[0]
    yc = lut[:, 1]
    xc = lut[:, 2]
    dy = jnp.arange(_S, dtype=jnp.int32)
    off = (dy[:, None] * _WP + dy[None, :]).reshape(_K)
    base = yc * _WP + xc
    idx = (off[:, None] + base[None, :]).reshape(_ROWS)
    idx2 = idx.reshape(_IDX_MAJOR, _IDX_MINOR).astype(jnp.int32)

    up128 = jnp.pad(unproj_range_xyz[0], ((0, 0), (0, _C - 4))).reshape(_Q, 128)

    eye4 = jnp.eye(_PPACK, dtype=f32)
    fc1p = jnp.pad(fc1_w.T, ((0, _C - 4), (0, 0)))
    fc1c = fc1p - jnp.mean(fc1p, axis=1, keepdims=True)
    fc1bd = jnp.kron(eye4, fc1c)
    b1c = jnp.tile(fc1_b - jnp.mean(fc1_b), _PPACK)
    lnw = jnp.tile(ln_w, _PPACK)
    lnb = jnp.tile(ln_b, _PPACK)
    gmat = jnp.kron(eye4, jnp.full((64, 1), 1.0 / 64, f32))
    bg4 = jnp.kron(eye4, jnp.ones((1, 64), f32))
    fc2bd = jnp.kron(eye4, fc2_w.T)
    b2t = jnp.tile(fc2_b, _PPACK)
    smat = jnp.kron(eye4, jnp.ones((_C, 1), f32))
    bs4 = jnp.kron(eye4, jnp.ones((1, _C), f32))
    convr3 = jnp.transpose(conv_w.reshape(_NCLASSES, _C, _K), (2, 1, 0))
    conv4 = jax.vmap(lambda m: jnp.kron(eye4, m))(convr3)
    cb = jnp.tile(conv_b, _PPACK)[None]

    gx, gp = _sc_gather(idx2, tx, tp)
    gx3 = gx.reshape(_K, _Q, 128)
    gp3 = gp.reshape(_K, _Q, 128)

    out = _tc_mlp(gx3, gp3, up128, fc1bd, b1c, lnw, lnb, gmat, bg4, fc2bd,
                  b2t, smat, bs4, conv4, cb)
    out = out.reshape(_Q, _PPACK, _NCLASSES).transpose(2, 0, 1).reshape(
        _NCLASSES, _N)
    return out[None]
